# baseline (speedup 1.0000x reference)
import functools

import jax
import jax.numpy as jnp
from jax import lax
from jax.experimental import pallas as pl
from jax.experimental.pallas import tpu as pltpu
from jax.experimental.pallas import tpu_sc as plsc

_TILE = 2048
_NW = 32


def _mm_body(x_ref, wt_ref, lg_ref):
    lg_ref[...] = jnp.dot(x_ref[...], wt_ref[...],
                          preferred_element_type=jnp.float32)


def _tc_logits(xf, wt):
    n, dim = xf.shape
    n_experts = wt.shape[1]
    grid = n // _TILE
    return pl.pallas_call(
        _mm_body,
        grid=(grid,),
        in_specs=[
            pl.BlockSpec((_TILE, dim), lambda i: (i, 0)),
            pl.BlockSpec((dim, n_experts), lambda i: (0, 0)),
        ],
        out_specs=pl.BlockSpec((_TILE, n_experts), lambda i: (i, 0)),
        out_shape=jax.ShapeDtypeStruct((n, n_experts), jnp.float32),
    )(xf, wt)


def _sc_route(lg):
    n, n_experts = lg.shape
    tpt = n // _NW
    mesh = plsc.VectorSubcoreMesh(core_axis_name="c", subcore_axis_name="s")

    @functools.partial(
        pl.kernel,
        mesh=mesh,
        compiler_params=pltpu.CompilerParams(
            needs_layout_passes=False, use_tc_tiling_on_sc=False),
        out_type=[
            jax.ShapeDtypeStruct((n,), jnp.int32),
            jax.ShapeDtypeStruct((n,), jnp.int32),
            jax.ShapeDtypeStruct((n,), jnp.float32),
            jax.ShapeDtypeStruct((n,), jnp.float32),
        ],
        scratch_types=[
            pltpu.VMEM((tpt, n_experts), jnp.float32),
            pltpu.VMEM((tpt,), jnp.int32),
            pltpu.VMEM((tpt,), jnp.int32),
            pltpu.VMEM((tpt,), jnp.float32),
            pltpu.VMEM((tpt,), jnp.float32),
        ],
    )
    def route(lg_hbm, i1_hbm, i2_hbm, w1_hbm, w2_hbm,
              lgv, i1v, i2v, w1v, w2v):
        lane = lax.iota(jnp.int32, 16)
        wid = lax.axis_index("s") * 2 + lax.axis_index("c")
        base = wid * tpt
        pltpu.sync_copy(lg_hbm.at[pl.ds(base, tpt)], lgv)

        def body(t, _):
            lgrow = lgv[t]
            ks, vs = plsc.sort_key_val(lgrow, lane, descending=True)
            r = jnp.exp(jnp.full((16,), ks[1] - ks[0], jnp.float32))
            w1 = 1.0 / (1.0 + r)
            w2 = r * w1
            tv = jnp.full((16,), t, jnp.int32)
            m0 = lane == 0
            m1m = lane == 1
            plsc.store_scatter(i1v, [tv], vs, mask=m0)
            plsc.store_scatter(i2v, [tv], vs, mask=m1m)
            plsc.store_scatter(w1v, [tv], w1, mask=m0)
            plsc.store_scatter(w2v, [tv], w2, mask=m0)
            return 0

        lax.fori_loop(0, tpt, body, 0)
        pltpu.sync_copy(i1v, i1_hbm.at[pl.ds(base, tpt)])
        pltpu.sync_copy(i2v, i2_hbm.at[pl.ds(base, tpt)])
        pltpu.sync_copy(w1v, w1_hbm.at[pl.ds(base, tpt)])
        pltpu.sync_copy(w2v, w2_hbm.at[pl.ds(base, tpt)])

    return route(lg)


def kernel(x, weight):
    n_experts, dim = weight.shape
    xf = x.reshape(-1, dim)
    lg = _tc_logits(xf, weight.T)
    i1, i2, w1, w2 = _sc_route(lg)
    idx = jnp.stack([i1, i2], axis=-1)
    w = jnp.stack([w1, w2], axis=-1)
    return idx, w
